# baseline (speedup 1.0000x reference)
import functools

import jax
import jax.numpy as jnp
from jax import lax
from jax.experimental import pallas as pl
from jax.experimental.pallas import tpu as pltpu
from jax.experimental.pallas import tpu_sc as plsc

N = 10000
E = 160000
E2 = 2 * E

NC = 2
NS = 16
NW = NC * NS
GEPW = E // NW
GOUTER = 200
GNOUT = GEPW // GOUTER
GKIDX = 40
GNG = GOUTER // GKIDX
EPW = E2 // NW
OUTER = 200
NOUT = EPW // OUTER
KIDX = 40
NG = OUTER // KIDX
NPAD = 10240
STRIPE = NPAD // NS

BE = 4000
BN = 2000
NBE = E2 // BE
NBE1 = E // BE
NBN = N // BN

F32 = jnp.float32


def _ln(x, g, b, eps=1e-5):
    m = jnp.mean(x, axis=-1, keepdims=True)
    v = jnp.mean((x - m) ** 2, axis=-1, keepdims=True)
    return (x - m) * lax.rsqrt(v + eps) * g + b


def _dot(a, b):
    return jnp.dot(a, b, preferred_element_type=F32)


def _full(shape):
    return pl.BlockSpec(shape, lambda i: tuple(0 for _ in shape))


def _sc_gather(table, src, dst):
    mesh = plsc.VectorSubcoreMesh(core_axis_name="c", subcore_axis_name="s")

    @functools.partial(
        pl.kernel,
        mesh=mesh,
        out_type=[jax.ShapeDtypeStruct((E, 128), F32),
                  jax.ShapeDtypeStruct((E, 128), F32)],
        scratch_types=[
            pltpu.VMEM((GEPW,), jnp.int32),
            pltpu.VMEM((GEPW,), jnp.int32),
            [pltpu.VMEM((GOUTER, 128), F32)] * 2,
            [pltpu.VMEM((GOUTER, 128), F32)] * 2,
            [pltpu.SemaphoreType.DMA] * 2,
        ],
    )
    def gather_k(t_hbm, src_hbm, dst_hbm, ts_hbm, td_hbm,
                 idxs, idxd, sbufs, dbufs, sems):
        wid = lax.axis_index("s") * NC + lax.axis_index("c")
        base = wid * GEPW
        pltpu.sync_copy(src_hbm.at[pl.ds(base, GEPW)], idxs)
        pltpu.sync_copy(dst_hbm.at[pl.ds(base, GEPW)], idxd)

        def fire(o, b):
            off = o * GOUTER
            for g in range(GNG):
                s = off + g * GKIDX
                pltpu.async_copy(t_hbm.at[idxs.at[pl.ds(s, GKIDX)]],
                                 sbufs[b].at[pl.ds(g * GKIDX, GKIDX)], sems[b])
                pltpu.async_copy(t_hbm.at[idxd.at[pl.ds(s, GKIDX)]],
                                 dbufs[b].at[pl.ds(g * GKIDX, GKIDX)], sems[b])

        def drain_write(o, b):
            pltpu.make_async_copy(
                ts_hbm.at[pl.ds(0, GOUTER)], sbufs[b], sems[b]).wait()
            pltpu.make_async_copy(
                ts_hbm.at[pl.ds(0, GOUTER)], dbufs[b], sems[b]).wait()
            off = base + o * GOUTER
            pltpu.sync_copy(sbufs[b], ts_hbm.at[pl.ds(off, GOUTER)])
            pltpu.sync_copy(dbufs[b], td_hbm.at[pl.ds(off, GOUTER)])

        fire(0, 0)

        def outer(o, carry):
            @pl.when(o % 2 == 0)
            def _():
                @pl.when(o + 1 < GNOUT)
                def _():
                    fire(o + 1, 1)
                drain_write(o, 0)

            @pl.when(o % 2 == 1)
            def _():
                @pl.when(o + 1 < GNOUT)
                def _():
                    fire(o + 1, 0)
                drain_write(o, 1)

            return carry

        lax.fori_loop(0, GNOUT, outer, 0)

    return gather_k(table, src, dst)


def _sc_scatter(e_core, col3, zeros_init):
    mesh = plsc.VectorSubcoreMesh(core_axis_name="c", subcore_axis_name="s")

    @functools.partial(
        pl.kernel,
        mesh=mesh,
        out_type=jax.ShapeDtypeStruct((NC, NPAD, 64), F32),
        scratch_types=[
            pltpu.VMEM_SHARED((NPAD, 64), F32),
            pltpu.VMEM((NOUT * NG, KIDX), jnp.int32),
            [pltpu.VMEM((OUTER, 64), F32)] * 2,
            [pltpu.SemaphoreType.DMA] * 2,
        ],
    )
    def scatter_k(ec_hbm, col3_hbm, zero_hbm, agg_hbm, shared, idx2,
                  vbufs, rsems):
        cid = lax.axis_index("c")
        sid = lax.axis_index("s")
        wid = sid * NC + cid
        base = wid * EPW
        pltpu.sync_copy(zero_hbm.at[pl.ds(sid * STRIPE, STRIPE)],
                        shared.at[pl.ds(sid * STRIPE, STRIPE)])
        pltpu.sync_copy(col3_hbm.at[wid], idx2)
        plsc.subcore_barrier()

        def fire_read(o, b):
            pltpu.async_copy(ec_hbm.at[pl.ds(base + o * OUTER, OUTER)],
                             vbufs[b], rsems[b])

        def scatter_chunk(o, b):
            pltpu.make_async_copy(
                ec_hbm.at[pl.ds(0, OUTER)], vbufs[b], rsems[b]).wait()
            for g in range(NG):
                pltpu.sync_copy(vbufs[b].at[pl.ds(g * KIDX, KIDX)],
                                shared.at[idx2.at[o * NG + g]], add=True)

        fire_read(0, 0)

        def outer(o, carry):
            @pl.when(o % 2 == 0)
            def _():
                @pl.when(o + 1 < NOUT)
                def _():
                    fire_read(o + 1, 1)
                scatter_chunk(o, 0)

            @pl.when(o % 2 == 1)
            def _():
                @pl.when(o + 1 < NOUT)
                def _():
                    fire_read(o + 1, 0)
                scatter_chunk(o, 1)

            return carry

        lax.fori_loop(0, NOUT, outer, 0)
        plsc.subcore_barrier()
        pltpu.sync_copy(shared.at[pl.ds(sid * STRIPE, STRIPE)],
                        agg_hbm.at[cid, pl.ds(sid * STRIPE, STRIPE)])

    return scatter_k(e_core, col3, zeros_init)


def _tc_pre_node(x, neW, neb, neg, nebe, W1r, W1c, NW1a,
                 ueb, ueg, uebe, emb1, W1u, nmb1, NW1u):
    def body(x_ref, neW_r, neb_r, neg_r, nebe_r, W1r_r, W1c_r, NW1a_r,
             ueb_r, ueg_r, uebe_r, emb1_r, W1u_r, nmb1_r, NW1u_r,
             t0_r, nrc_r, vc_r, uenc_r, eb_r, nb_r):
        i = pl.program_id(0)
        xb = x_ref[...]
        h = jnp.maximum(_dot(xb, neW_r[...]) + neb_r[0:1, :], 0.0)
        xe = _ln(h, neg_r[0:1, :], nebe_r[0:1, :])
        t0_r[...] = jnp.concatenate(
            [_dot(xe, W1r_r[...]), _dot(xe, W1c_r[...])], axis=1)
        nrc_r[...] = _dot(xe, NW1a_r[...])
        mf = (xb[:, 0:1] > xb[:, 1:2]).astype(F32)

        @pl.when(i == 0)
        def _():
            vc_r[...] = jnp.zeros_like(vc_r)

        vc_r[...] += jnp.broadcast_to(jnp.sum(mf), (8, 64))

        @pl.when(i == NBN - 1)
        def _():
            ue = _ln(jnp.maximum(ueb_r[0:1, :], 0.0),
                     ueg_r[0:1, :], uebe_r[0:1, :])
            uenc_r[...] = jnp.broadcast_to(ue, (8, 32))
            eb_r[...] = jnp.broadcast_to(
                emb1_r[0:1, :] + _dot(ue, W1u_r[...]), (8, 64))
            nb_r[...] = jnp.broadcast_to(
                nmb1_r[0:1, :] + _dot(ue, NW1u_r[...]), (8, 64))

    return pl.pallas_call(
        body,
        grid=(NBN,),
        in_specs=[
            pl.BlockSpec((BN, 2), lambda i: (i, 0)),
            _full((2, 32)), _full((8, 32)), _full((8, 32)), _full((8, 32)),
            _full((32, 64)), _full((32, 64)), _full((32, 64)),
            _full((8, 32)), _full((8, 32)), _full((8, 32)),
            _full((8, 64)), _full((32, 64)), _full((8, 64)), _full((32, 64)),
        ],
        out_specs=[
            pl.BlockSpec((BN, 128), lambda i: (i, 0)),
            pl.BlockSpec((BN, 64), lambda i: (i, 0)),
            pl.BlockSpec((8, 64), lambda i: (0, 0)),
            pl.BlockSpec((8, 32), lambda i: (0, 0)),
            pl.BlockSpec((8, 64), lambda i: (0, 0)),
            pl.BlockSpec((8, 64), lambda i: (0, 0)),
        ],
        out_shape=[
            jax.ShapeDtypeStruct((N, 128), F32),
            jax.ShapeDtypeStruct((N, 64), F32),
            jax.ShapeDtypeStruct((8, 64), F32),
            jax.ShapeDtypeStruct((8, 32), F32),
            jax.ShapeDtypeStruct((8, 64), F32),
            jax.ShapeDtypeStruct((8, 64), F32),
        ],
    )(x, neW, neb, neg, nebe, W1r, W1c, NW1a,
      ueb, ueg, uebe, emb1, W1u, nmb1, NW1u)


def _tc_pre_edge(edge_attr, eeW, eeb, eeg, eebe, W1e):
    def body(ea_ref, eeW_r, eeb_r, eeg_r, eebe_r, W1e_r, ec_r):
        h = jnp.maximum(_dot(ea_ref[...], eeW_r[...]) + eeb_r[0:1, :], 0.0)
        ee = _ln(h, eeg_r[0:1, :], eebe_r[0:1, :])
        ec_r[...] = _dot(ee, W1e_r[...])

    return pl.pallas_call(
        body,
        grid=(NBE1,),
        in_specs=[
            pl.BlockSpec((BE, 2), lambda i: (i, 0)),
            _full((2, 32)), _full((8, 32)), _full((8, 32)), _full((8, 32)),
            _full((32, 64)),
        ],
        out_specs=pl.BlockSpec((BE, 64), lambda i: (i, 0)),
        out_shape=jax.ShapeDtypeStruct((E, 64), F32),
    )(edge_attr, eeW, eeb, eeg, eebe, W1e)


def _tc_edge(prev, EC, Ts, Td, W1c, W2, eb, b2, g, be, with_prev):
    def body(*refs):
        if with_prev:
            (prev_r, ec_r, ts_r, td_r, W1c_r, W2_r, eb_r, b2_r, g_r, be_r,
             out_r, esum_r) = refs
        else:
            (ec_r, ts_r, td_r, W2_r, eb_r, b2_r, g_r, be_r,
             out_r, esum_r) = refs
        i = pl.program_id(0)
        ec = ec_r[...]
        ts = ts_r[...]
        td = td_r[...]
        bias = eb_r[0:1, :]
        pre_f = ec + ts[:, 0:64] + td[:, 64:128] + bias
        pre_b = ec + td[:, 0:64] + ts[:, 64:128] + bias
        pre = jnp.concatenate([pre_f, pre_b], axis=0)
        if with_prev:
            prev2 = jnp.concatenate([prev_r[0], prev_r[1]], axis=0)
            pre = pre + _dot(prev2, W1c_r[...])
        h = jnp.maximum(pre, 0.0)
        ecore = _ln(_dot(h, W2_r[...]) + b2_r[0:1, :], g_r[0:1, :], be_r[0:1, :])
        out_r[0] = ecore[0:BE]
        out_r[1] = ecore[BE:2 * BE]

        @pl.when(i == 0)
        def _():
            esum_r[...] = jnp.zeros_like(esum_r)

        esum_r[...] += jnp.broadcast_to(
            jnp.sum(ecore, axis=0, keepdims=True), (8, 64))

    eblk64 = pl.BlockSpec((BE, 64), lambda i: (i, 0))
    eblk128 = pl.BlockSpec((BE, 128), lambda i: (i, 0))
    in_specs = [eblk64, eblk128, eblk128,
                _full((64, 64)), _full((8, 64)), _full((8, 64)),
                _full((8, 64)), _full((8, 64))]
    args = [EC, Ts, Td, W2, eb, b2, g, be]
    if with_prev:
        in_specs = [pl.BlockSpec((2, BE, 64), lambda i: (0, i, 0))] \
            + in_specs[:3] + [_full((64, 64))] + in_specs[3:]
        args = [prev, EC, Ts, Td, W1c, W2, eb, b2, g, be]

    return pl.pallas_call(
        body,
        grid=(NBE1,),
        in_specs=in_specs,
        out_specs=[
            pl.BlockSpec((2, BE, 64), lambda i: (0, i, 0)),
            pl.BlockSpec((8, 64), lambda i: (0, 0)),
        ],
        out_shape=[
            jax.ShapeDtypeStruct((2, E, 64), F32),
            jax.ShapeDtypeStruct((8, 64), F32),
        ],
    )(*args)


def _tc_node(x, NRc, xcp, agg2, NW1x, NW1agg, nmW2, nb, nmb2, nmg, nmbe,
             T0, EW1xr, EW1xc,
             esum, vc, uenc, ucore, G0, G1, G2, G3, gmb1, gmW2, gmb2,
             gmg, gmbe, emb1, W1u, W1u2, nmb1, NW1u, NW1u2, with_prev):
    def body(*refs):
        if with_prev:
            (x_r, nrc_r, xcp_r, agg_r, NW1x_r, NW1agg_r, nmW2_r, nb_r,
             nmb2_r, nmg_r, nmbe_r, t0_r, EW1xr_r, EW1xc_r,
             es_r, vc_r, ue_r, uc_r, G0_r, G1_r, G2_r, G3_r, gmb1_r,
             gmW2_r, gmb2_r, gmg_r, gmbe_r, emb1_r, W1u_r, W1u2_r,
             nmb1_r, NW1u_r, NW1u2_r,
             xc_r, t_r, vsum_r, uo_r, ebo_r, nbo_r) = refs
        else:
            (x_r, nrc_r, agg_r, NW1agg_r, nmW2_r, nb_r,
             nmb2_r, nmg_r, nmbe_r, t0_r, EW1xr_r, EW1xc_r,
             es_r, vc_r, ue_r, G0_r, G2_r, G3_r, gmb1_r,
             gmW2_r, gmb2_r, gmg_r, gmbe_r, emb1_r, W1u_r, W1u2_r,
             nmb1_r, NW1u_r, NW1u2_r,
             xc_r, t_r, vsum_r, uo_r, ebo_r, nbo_r) = refs
        i = pl.program_id(0)
        agg = agg_r[0] + agg_r[1]
        pre = nrc_r[...] + _dot(agg, NW1agg_r[...]) + nb_r[0:1, :]
        if with_prev:
            pre = pre + _dot(xcp_r[...], NW1x_r[...])
        h = jnp.maximum(pre, 0.0)
        xc = _ln(_dot(h, nmW2_r[...]) + nmb2_r[0:1, :],
                 nmg_r[0:1, :], nmbe_r[0:1, :])
        xc_r[...] = xc
        t0 = t0_r[...]
        t_r[...] = t0 + jnp.concatenate(
            [_dot(xc, EW1xr_r[...]), _dot(xc, EW1xc_r[...])], axis=1)
        xb = x_r[...]
        mf = (xb[:, 0:1] > xb[:, 1:2]).astype(F32)

        @pl.when(i == 0)
        def _():
            vsum_r[...] = jnp.zeros_like(vsum_r)

        vsum_r[...] += jnp.broadcast_to(
            jnp.sum(xc * mf, axis=0, keepdims=True), (8, 64))

        @pl.when(i == NBN - 1)
        def _():
            ue = ue_r[0:1, :]
            ea = es_r[0:1, :] * (1.0 / E2)
            cnt = vc_r[0:1, 0:1]
            na = vsum_r[0:1, :] / jnp.maximum(cnt, 1.0)
            upre = (_dot(ue, G0_r[...]) + _dot(ea, G2_r[...])
                    + _dot(na, G3_r[...]) + gmb1_r[0:1, :])
            if with_prev:
                upre = upre + _dot(uc_r[0:1, :], G1_r[...])
            uh = jnp.maximum(upre, 0.0)
            uc = _ln(_dot(uh, gmW2_r[...]) + gmb2_r[0:1, :],
                     gmg_r[0:1, :], gmbe_r[0:1, :])
            uo_r[...] = jnp.broadcast_to(uc, (8, 32))
            ebo_r[...] = jnp.broadcast_to(
                emb1_r[0:1, :] + _dot(ue, W1u_r[...]) + _dot(uc, W1u2_r[...]),
                (8, 64))
            nbo_r[...] = jnp.broadcast_to(
                nmb1_r[0:1, :] + _dot(ue, NW1u_r[...]) + _dot(uc, NW1u2_r[...]),
                (8, 64))

    nblk = lambda i: (i, 0)
    in_specs = [pl.BlockSpec((BN, 2), nblk),
                pl.BlockSpec((BN, 64), nblk)]
    args = [x, NRc]
    if with_prev:
        in_specs += [pl.BlockSpec((BN, 64), nblk)]
        args += [xcp]
    in_specs += [pl.BlockSpec((NC, BN, 64), lambda i: (0, i, 0))]
    args += [agg2]
    if with_prev:
        in_specs += [_full((64, 64))]
        args += [NW1x]
    in_specs += [_full((64, 64)), _full((64, 64)), _full((8, 64)),
                 _full((8, 64)), _full((8, 64)), _full((8, 64)),
                 pl.BlockSpec((BN, 128), nblk),
                 _full((64, 64)), _full((64, 64))]
    args += [NW1agg, nmW2, nb, nmb2, nmg, nmbe, T0, EW1xr, EW1xc]
    in_specs += [_full((8, 64)), _full((8, 64)), _full((8, 32))]
    args += [esum, vc, uenc]
    if with_prev:
        in_specs += [_full((8, 32))]
        args += [ucore]
    in_specs += [_full((32, 64))]
    args += [G0]
    if with_prev:
        in_specs += [_full((32, 64))]
        args += [G1]
    in_specs += [_full((64, 64)), _full((64, 64)), _full((8, 64)),
                 _full((64, 32)), _full((8, 32)), _full((8, 32)),
                 _full((8, 32)), _full((8, 64)), _full((32, 64)),
                 _full((32, 64)), _full((8, 64)), _full((32, 64)),
                 _full((32, 64))]
    args += [G2, G3, gmb1, gmW2, gmb2, gmg, gmbe, emb1, W1u, W1u2,
             nmb1, NW1u, NW1u2]

    return pl.pallas_call(
        body,
        grid=(NBN,),
        in_specs=in_specs,
        out_specs=[
            pl.BlockSpec((BN, 64), nblk),
            pl.BlockSpec((BN, 128), nblk),
            pl.BlockSpec((8, 64), lambda i: (0, 0)),
            pl.BlockSpec((8, 32), lambda i: (0, 0)),
            pl.BlockSpec((8, 64), lambda i: (0, 0)),
            pl.BlockSpec((8, 64), lambda i: (0, 0)),
        ],
        out_shape=[
            jax.ShapeDtypeStruct((N, 64), F32),
            jax.ShapeDtypeStruct((N, 128), F32),
            jax.ShapeDtypeStruct((8, 64), F32),
            jax.ShapeDtypeStruct((8, 32), F32),
            jax.ShapeDtypeStruct((8, 64), F32),
            jax.ShapeDtypeStruct((8, 64), F32),
        ],
    )(*args)


def _tc_decode(x_core, decW, decb, decg, decbe, qW, qb):
    def body(xc_r, decW_r, decb_r, decg_r, decbe_r, qW_r, qb_r, qs_r):
        h = jnp.maximum(_dot(xc_r[...], decW_r[...]) + decb_r[0:1, :], 0.0)
        nh = _ln(h, decg_r[0:1, :], decbe_r[0:1, :])
        qs_r[...] = _dot(nh, qW_r[...]) + qb_r[0:1, :]

    return pl.pallas_call(
        body,
        grid=(NBN,),
        in_specs=[
            pl.BlockSpec((BN, 64), lambda i: (i, 0)),
            _full((64, 32)), _full((8, 32)), _full((8, 32)), _full((8, 32)),
            _full((32, 2)), _full((8, 2)),
        ],
        out_specs=pl.BlockSpec((BN, 2), lambda i: (i, 0)),
        out_shape=jax.ShapeDtypeStruct((N, 2), F32),
    )(x_core, decW, decb, decg, decbe, qW, qb)


def kernel(x, edge_index, edge_attr, params):
    p = params
    src = edge_index[0].astype(jnp.int32)
    dst = edge_index[1].astype(jnp.int32)
    col3 = jnp.concatenate([dst, src]).reshape(NW, NOUT * NG, KIDX)

    def tile8(v):
        return jnp.tile(v.reshape(1, -1), (8, 1))

    W1 = p['em_W1']
    NW1 = p['nm_W1']
    GW = p['gm_W1']

    T0, NRc, vc, uenc, eb, nb = _tc_pre_node(
        x, p['ne_W'], tile8(p['ne_b']), tile8(p['ne_g']), tile8(p['ne_be']),
        W1[96:128], W1[192:224], NW1[0:32],
        tile8(p['ue_b']), tile8(p['ue_g']), tile8(p['ue_be']),
        tile8(p['em_b1']), W1[288:320], tile8(p['nm_b1']), NW1[160:192])
    EC = _tc_pre_edge(
        edge_attr, p['ee_W'], tile8(p['ee_b']), tile8(p['ee_g']),
        tile8(p['ee_be']), W1[0:32])

    zeros_init = jnp.zeros((NPAD, 64), F32)
    emb2 = tile8(p['em_b2'])
    emg = tile8(p['em_g'])
    embe = tile8(p['em_be'])
    nmb2 = tile8(p['nm_b2'])
    nmg = tile8(p['nm_g'])
    nmbe = tile8(p['nm_be'])

    e_core = None
    x_core = None
    u_core = None
    T = T0
    for r in range(4):
        Ts, Td = _sc_gather(T, src, dst)
        e_core, esum = _tc_edge(
            e_core, EC, Ts, Td, W1[32:96], p['em_W2'], eb, emb2, emg, embe,
            with_prev=(r > 0))
        agg2 = _sc_scatter(e_core.reshape(E2, 64), col3, zeros_init)
        x_core, T, vsum, u_core, eb, nb = _tc_node(
            x, NRc, x_core, agg2, NW1[32:96], NW1[96:160], p['nm_W2'], nb,
            nmb2, nmg, nmbe, T0, W1[128:192], W1[224:288],
            esum, vc, uenc, u_core, GW[0:32], GW[32:64], GW[64:128],
            GW[128:192], tile8(p['gm_b1']), p['gm_W2'], tile8(p['gm_b2']),
            tile8(p['gm_g']), tile8(p['gm_be']), tile8(p['em_b1']),
            W1[288:320], W1[320:352], tile8(p['nm_b1']), NW1[160:192],
            NW1[192:224], with_prev=(r > 0))

    qs = _tc_decode(x_core, p['dec_W'], tile8(p['dec_b']), tile8(p['dec_g']),
                    tile8(p['dec_be']), p['q_W'], tile8(p['q_b']))
    var_mask = x[:, 0] > x[:, 1]
    return qs, var_mask


# speedup vs baseline: 1.1075x; 1.1075x over previous
import functools

import jax
import jax.numpy as jnp
from jax import lax
from jax.experimental import pallas as pl
from jax.experimental.pallas import tpu as pltpu
from jax.experimental.pallas import tpu_sc as plsc

N = 10000
E = 160000
E2 = 2 * E

NC = 2
NS = 16
NW = NC * NS
GEPW = E // NW
GOUTER = 200
GNOUT = GEPW // GOUTER
GKIDX = 40
GNG = GOUTER // GKIDX
EPW = E2 // NW
OUTER = 200
NOUT = EPW // OUTER
KIDX = 40
NG = OUTER // KIDX
NPAD = 10240
STRIPE = NPAD // NS

BE = 4000
BN = 2000
NBE = E2 // BE
NBE1 = E // BE
NBN = N // BN

F32 = jnp.float32


def _ln(x, g, b, eps=1e-5):
    m = jnp.mean(x, axis=-1, keepdims=True)
    v = jnp.mean((x - m) ** 2, axis=-1, keepdims=True)
    return (x - m) * lax.rsqrt(v + eps) * g + b


def _dot(a, b):
    return jnp.dot(a, b, preferred_element_type=F32)


def _full(shape):
    return pl.BlockSpec(shape, lambda i: tuple(0 for _ in shape))


def _sc_gather(table, src, dst):
    mesh = plsc.VectorSubcoreMesh(core_axis_name="c", subcore_axis_name="s")

    @functools.partial(
        pl.kernel,
        mesh=mesh,
        out_type=jax.ShapeDtypeStruct((E, 128), F32),
        scratch_types=[
            pltpu.VMEM((GEPW,), jnp.int32),
            pltpu.VMEM((GEPW,), jnp.int32),
            [pltpu.VMEM((GOUTER, 128), F32)] * 2,
            [pltpu.VMEM((GOUTER, 128), F32)] * 2,
            [pltpu.SemaphoreType.DMA] * 2,
        ],
    )
    def gather_k(t_hbm, src_hbm, dst_hbm, g_hbm,
                 idxs, idxd, sbufs, dbufs, sems):
        wid = lax.axis_index("s") * NC + lax.axis_index("c")
        base = wid * GEPW
        pltpu.sync_copy(src_hbm.at[pl.ds(base, GEPW)], idxs)
        pltpu.sync_copy(dst_hbm.at[pl.ds(base, GEPW)], idxd)

        def fire(o, b):
            off = o * GOUTER
            for g in range(GNG):
                s = off + g * GKIDX
                pltpu.async_copy(t_hbm.at[idxs.at[pl.ds(s, GKIDX)]],
                                 sbufs[b].at[pl.ds(g * GKIDX, GKIDX)], sems[b])
                pltpu.async_copy(t_hbm.at[idxd.at[pl.ds(s, GKIDX)]],
                                 dbufs[b].at[pl.ds(g * GKIDX, GKIDX)], sems[b])

        def drain_sum_write(o, b):
            pltpu.make_async_copy(
                g_hbm.at[pl.ds(0, GOUTER)], sbufs[b], sems[b]).wait()
            pltpu.make_async_copy(
                g_hbm.at[pl.ds(0, GOUTER)], dbufs[b], sems[b]).wait()
            sb = sbufs[b]
            db = dbufs[b]

            def addrow(e, carry):
                for j in range(4):
                    db[e, pl.ds(64 + j * 16, 16)] += sb[e, pl.ds(j * 16, 16)]
                    db[e, pl.ds(j * 16, 16)] += sb[e, pl.ds(64 + j * 16, 16)]
                return carry

            lax.fori_loop(0, GOUTER, addrow, 0)
            off = base + o * GOUTER
            pltpu.sync_copy(db, g_hbm.at[pl.ds(off, GOUTER)])

        fire(0, 0)

        def outer(o, carry):
            @pl.when(o % 2 == 0)
            def _():
                @pl.when(o + 1 < GNOUT)
                def _():
                    fire(o + 1, 1)
                drain_sum_write(o, 0)

            @pl.when(o % 2 == 1)
            def _():
                @pl.when(o + 1 < GNOUT)
                def _():
                    fire(o + 1, 0)
                drain_sum_write(o, 1)

            return carry

        lax.fori_loop(0, GNOUT, outer, 0)

    return gather_k(table, src, dst)


def _sc_scatter(e_core, col3, zeros_init):
    mesh = plsc.VectorSubcoreMesh(core_axis_name="c", subcore_axis_name="s")

    @functools.partial(
        pl.kernel,
        mesh=mesh,
        out_type=jax.ShapeDtypeStruct((NC, NPAD, 64), F32),
        scratch_types=[
            pltpu.VMEM_SHARED((NPAD, 64), F32),
            pltpu.VMEM((NOUT * NG, KIDX), jnp.int32),
            [pltpu.VMEM((OUTER, 64), F32)] * 2,
            [pltpu.SemaphoreType.DMA] * 2,
        ],
    )
    def scatter_k(ec_hbm, col3_hbm, zero_hbm, agg_hbm, shared, idx2,
                  vbufs, rsems):
        cid = lax.axis_index("c")
        sid = lax.axis_index("s")
        wid = sid * NC + cid
        base = wid * EPW
        pltpu.sync_copy(zero_hbm.at[pl.ds(sid * STRIPE, STRIPE)],
                        shared.at[pl.ds(sid * STRIPE, STRIPE)])
        pltpu.sync_copy(col3_hbm.at[wid], idx2)
        plsc.subcore_barrier()

        def fire_read(o, b):
            pltpu.async_copy(ec_hbm.at[pl.ds(base + o * OUTER, OUTER)],
                             vbufs[b], rsems[b])

        def scatter_chunk(o, b):
            pltpu.make_async_copy(
                ec_hbm.at[pl.ds(0, OUTER)], vbufs[b], rsems[b]).wait()
            for g in range(NG):
                pltpu.sync_copy(vbufs[b].at[pl.ds(g * KIDX, KIDX)],
                                shared.at[idx2.at[o * NG + g]], add=True)

        fire_read(0, 0)

        def outer(o, carry):
            @pl.when(o % 2 == 0)
            def _():
                @pl.when(o + 1 < NOUT)
                def _():
                    fire_read(o + 1, 1)
                scatter_chunk(o, 0)

            @pl.when(o % 2 == 1)
            def _():
                @pl.when(o + 1 < NOUT)
                def _():
                    fire_read(o + 1, 0)
                scatter_chunk(o, 1)

            return carry

        lax.fori_loop(0, NOUT, outer, 0)
        plsc.subcore_barrier()
        pltpu.sync_copy(shared.at[pl.ds(sid * STRIPE, STRIPE)],
                        agg_hbm.at[cid, pl.ds(sid * STRIPE, STRIPE)])

    return scatter_k(e_core, col3, zeros_init)


def _tc_pre_node(x, neW, neb, neg, nebe, W1r, W1c, NW1a,
                 ueb, ueg, uebe, emb1, W1u, nmb1, NW1u):
    def body(x_ref, neW_r, neb_r, neg_r, nebe_r, W1r_r, W1c_r, NW1a_r,
             ueb_r, ueg_r, uebe_r, emb1_r, W1u_r, nmb1_r, NW1u_r,
             t0_r, nrc_r, vc_r, uenc_r, eb_r, nb_r):
        i = pl.program_id(0)
        xb = x_ref[...]
        h = jnp.maximum(_dot(xb, neW_r[...]) + neb_r[0:1, :], 0.0)
        xe = _ln(h, neg_r[0:1, :], nebe_r[0:1, :])
        t0_r[...] = jnp.concatenate(
            [_dot(xe, W1r_r[...]), _dot(xe, W1c_r[...])], axis=1)
        nrc_r[...] = _dot(xe, NW1a_r[...])
        mf = (xb[:, 0:1] > xb[:, 1:2]).astype(F32)

        @pl.when(i == 0)
        def _():
            vc_r[...] = jnp.zeros_like(vc_r)

        vc_r[...] += jnp.broadcast_to(jnp.sum(mf), (8, 64))

        @pl.when(i == NBN - 1)
        def _():
            ue = _ln(jnp.maximum(ueb_r[0:1, :], 0.0),
                     ueg_r[0:1, :], uebe_r[0:1, :])
            uenc_r[...] = jnp.broadcast_to(ue, (8, 32))
            eb_r[...] = jnp.broadcast_to(
                emb1_r[0:1, :] + _dot(ue, W1u_r[...]), (8, 64))
            nb_r[...] = jnp.broadcast_to(
                nmb1_r[0:1, :] + _dot(ue, NW1u_r[...]), (8, 64))

    return pl.pallas_call(
        body,
        grid=(NBN,),
        in_specs=[
            pl.BlockSpec((BN, 2), lambda i: (i, 0)),
            _full((2, 32)), _full((8, 32)), _full((8, 32)), _full((8, 32)),
            _full((32, 64)), _full((32, 64)), _full((32, 64)),
            _full((8, 32)), _full((8, 32)), _full((8, 32)),
            _full((8, 64)), _full((32, 64)), _full((8, 64)), _full((32, 64)),
        ],
        out_specs=[
            pl.BlockSpec((BN, 128), lambda i: (i, 0)),
            pl.BlockSpec((BN, 64), lambda i: (i, 0)),
            pl.BlockSpec((8, 64), lambda i: (0, 0)),
            pl.BlockSpec((8, 32), lambda i: (0, 0)),
            pl.BlockSpec((8, 64), lambda i: (0, 0)),
            pl.BlockSpec((8, 64), lambda i: (0, 0)),
        ],
        out_shape=[
            jax.ShapeDtypeStruct((N, 128), F32),
            jax.ShapeDtypeStruct((N, 64), F32),
            jax.ShapeDtypeStruct((8, 64), F32),
            jax.ShapeDtypeStruct((8, 32), F32),
            jax.ShapeDtypeStruct((8, 64), F32),
            jax.ShapeDtypeStruct((8, 64), F32),
        ],
    )(x, neW, neb, neg, nebe, W1r, W1c, NW1a,
      ueb, ueg, uebe, emb1, W1u, nmb1, NW1u)


def _tc_pre_edge(edge_attr, eeW, eeb, eeg, eebe, W1e):
    def body(ea_ref, eeW_r, eeb_r, eeg_r, eebe_r, W1e_r, ec_r):
        h = jnp.maximum(_dot(ea_ref[...], eeW_r[...]) + eeb_r[0:1, :], 0.0)
        ee = _ln(h, eeg_r[0:1, :], eebe_r[0:1, :])
        ec_r[...] = _dot(ee, W1e_r[...])

    return pl.pallas_call(
        body,
        grid=(NBE1,),
        in_specs=[
            pl.BlockSpec((BE, 2), lambda i: (i, 0)),
            _full((2, 32)), _full((8, 32)), _full((8, 32)), _full((8, 32)),
            _full((32, 64)),
        ],
        out_specs=pl.BlockSpec((BE, 64), lambda i: (i, 0)),
        out_shape=jax.ShapeDtypeStruct((E, 64), F32),
    )(edge_attr, eeW, eeb, eeg, eebe, W1e)


def _tc_edge(prev, EC, G, W1c, W2, eb, b2, g, be, with_prev):
    def body(*refs):
        if with_prev:
            (prev_r, ec_r, g_r2, W1c_r, W2_r, eb_r, b2_r, g_r, be_r,
             out_r, esum_r) = refs
        else:
            (ec_r, g_r2, W2_r, eb_r, b2_r, g_r, be_r,
             out_r, esum_r) = refs
        i = pl.program_id(0)
        ec = ec_r[...]
        gg = g_r2[...]
        bias = eb_r[0:1, :]
        pre_f = ec + gg[:, 64:128] + bias
        pre_b = ec + gg[:, 0:64] + bias
        pre = jnp.concatenate([pre_f, pre_b], axis=0)
        if with_prev:
            prev2 = jnp.concatenate([prev_r[0], prev_r[1]], axis=0)
            pre = pre + _dot(prev2, W1c_r[...])
        h = jnp.maximum(pre, 0.0)
        ecore = _ln(_dot(h, W2_r[...]) + b2_r[0:1, :], g_r[0:1, :], be_r[0:1, :])
        out_r[0] = ecore[0:BE]
        out_r[1] = ecore[BE:2 * BE]

        @pl.when(i == 0)
        def _():
            esum_r[...] = jnp.zeros_like(esum_r)

        esum_r[...] += jnp.broadcast_to(
            jnp.sum(ecore, axis=0, keepdims=True), (8, 64))

    eblk64 = pl.BlockSpec((BE, 64), lambda i: (i, 0))
    eblk128 = pl.BlockSpec((BE, 128), lambda i: (i, 0))
    in_specs = [eblk64, eblk128,
                _full((64, 64)), _full((8, 64)), _full((8, 64)),
                _full((8, 64)), _full((8, 64))]
    args = [EC, G, W2, eb, b2, g, be]
    if with_prev:
        in_specs = [pl.BlockSpec((2, BE, 64), lambda i: (0, i, 0))] \
            + in_specs[:2] + [_full((64, 64))] + in_specs[2:]
        args = [prev, EC, G, W1c, W2, eb, b2, g, be]

    return pl.pallas_call(
        body,
        grid=(NBE1,),
        in_specs=in_specs,
        out_specs=[
            pl.BlockSpec((2, BE, 64), lambda i: (0, i, 0)),
            pl.BlockSpec((8, 64), lambda i: (0, 0)),
        ],
        out_shape=[
            jax.ShapeDtypeStruct((2, E, 64), F32),
            jax.ShapeDtypeStruct((8, 64), F32),
        ],
    )(*args)


def _tc_node(x, NRc, xcp, agg2, NW1x, NW1agg, nmW2, nb, nmb2, nmg, nmbe,
             T0, EW1xr, EW1xc,
             esum, vc, uenc, ucore, G0, G1, G2, G3, gmb1, gmW2, gmb2,
             gmg, gmbe, emb1, W1u, W1u2, nmb1, NW1u, NW1u2, with_prev):
    def body(*refs):
        if with_prev:
            (x_r, nrc_r, xcp_r, agg_r, NW1x_r, NW1agg_r, nmW2_r, nb_r,
             nmb2_r, nmg_r, nmbe_r, t0_r, EW1xr_r, EW1xc_r,
             es_r, vc_r, ue_r, uc_r, G0_r, G1_r, G2_r, G3_r, gmb1_r,
             gmW2_r, gmb2_r, gmg_r, gmbe_r, emb1_r, W1u_r, W1u2_r,
             nmb1_r, NW1u_r, NW1u2_r,
             xc_r, t_r, vsum_r, uo_r, ebo_r, nbo_r) = refs
        else:
            (x_r, nrc_r, agg_r, NW1agg_r, nmW2_r, nb_r,
             nmb2_r, nmg_r, nmbe_r, t0_r, EW1xr_r, EW1xc_r,
             es_r, vc_r, ue_r, G0_r, G2_r, G3_r, gmb1_r,
             gmW2_r, gmb2_r, gmg_r, gmbe_r, emb1_r, W1u_r, W1u2_r,
             nmb1_r, NW1u_r, NW1u2_r,
             xc_r, t_r, vsum_r, uo_r, ebo_r, nbo_r) = refs
        i = pl.program_id(0)
        agg = agg_r[0] + agg_r[1]
        pre = nrc_r[...] + _dot(agg, NW1agg_r[...]) + nb_r[0:1, :]
        if with_prev:
            pre = pre + _dot(xcp_r[...], NW1x_r[...])
        h = jnp.maximum(pre, 0.0)
        xc = _ln(_dot(h, nmW2_r[...]) + nmb2_r[0:1, :],
                 nmg_r[0:1, :], nmbe_r[0:1, :])
        xc_r[...] = xc
        t0 = t0_r[...]
        t_r[...] = t0 + jnp.concatenate(
            [_dot(xc, EW1xr_r[...]), _dot(xc, EW1xc_r[...])], axis=1)
        xb = x_r[...]
        mf = (xb[:, 0:1] > xb[:, 1:2]).astype(F32)

        @pl.when(i == 0)
        def _():
            vsum_r[...] = jnp.zeros_like(vsum_r)

        vsum_r[...] += jnp.broadcast_to(
            jnp.sum(xc * mf, axis=0, keepdims=True), (8, 64))

        @pl.when(i == NBN - 1)
        def _():
            ue = ue_r[0:1, :]
            ea = es_r[0:1, :] * (1.0 / E2)
            cnt = vc_r[0:1, 0:1]
            na = vsum_r[0:1, :] / jnp.maximum(cnt, 1.0)
            upre = (_dot(ue, G0_r[...]) + _dot(ea, G2_r[...])
                    + _dot(na, G3_r[...]) + gmb1_r[0:1, :])
            if with_prev:
                upre = upre + _dot(uc_r[0:1, :], G1_r[...])
            uh = jnp.maximum(upre, 0.0)
            uc = _ln(_dot(uh, gmW2_r[...]) + gmb2_r[0:1, :],
                     gmg_r[0:1, :], gmbe_r[0:1, :])
            uo_r[...] = jnp.broadcast_to(uc, (8, 32))
            ebo_r[...] = jnp.broadcast_to(
                emb1_r[0:1, :] + _dot(ue, W1u_r[...]) + _dot(uc, W1u2_r[...]),
                (8, 64))
            nbo_r[...] = jnp.broadcast_to(
                nmb1_r[0:1, :] + _dot(ue, NW1u_r[...]) + _dot(uc, NW1u2_r[...]),
                (8, 64))

    nblk = lambda i: (i, 0)
    in_specs = [pl.BlockSpec((BN, 2), nblk),
                pl.BlockSpec((BN, 64), nblk)]
    args = [x, NRc]
    if with_prev:
        in_specs += [pl.BlockSpec((BN, 64), nblk)]
        args += [xcp]
    in_specs += [pl.BlockSpec((NC, BN, 64), lambda i: (0, i, 0))]
    args += [agg2]
    if with_prev:
        in_specs += [_full((64, 64))]
        args += [NW1x]
    in_specs += [_full((64, 64)), _full((64, 64)), _full((8, 64)),
                 _full((8, 64)), _full((8, 64)), _full((8, 64)),
                 pl.BlockSpec((BN, 128), nblk),
                 _full((64, 64)), _full((64, 64))]
    args += [NW1agg, nmW2, nb, nmb2, nmg, nmbe, T0, EW1xr, EW1xc]
    in_specs += [_full((8, 64)), _full((8, 64)), _full((8, 32))]
    args += [esum, vc, uenc]
    if with_prev:
        in_specs += [_full((8, 32))]
        args += [ucore]
    in_specs += [_full((32, 64))]
    args += [G0]
    if with_prev:
        in_specs += [_full((32, 64))]
        args += [G1]
    in_specs += [_full((64, 64)), _full((64, 64)), _full((8, 64)),
                 _full((64, 32)), _full((8, 32)), _full((8, 32)),
                 _full((8, 32)), _full((8, 64)), _full((32, 64)),
                 _full((32, 64)), _full((8, 64)), _full((32, 64)),
                 _full((32, 64))]
    args += [G2, G3, gmb1, gmW2, gmb2, gmg, gmbe, emb1, W1u, W1u2,
             nmb1, NW1u, NW1u2]

    return pl.pallas_call(
        body,
        grid=(NBN,),
        in_specs=in_specs,
        out_specs=[
            pl.BlockSpec((BN, 64), nblk),
            pl.BlockSpec((BN, 128), nblk),
            pl.BlockSpec((8, 64), lambda i: (0, 0)),
            pl.BlockSpec((8, 32), lambda i: (0, 0)),
            pl.BlockSpec((8, 64), lambda i: (0, 0)),
            pl.BlockSpec((8, 64), lambda i: (0, 0)),
        ],
        out_shape=[
            jax.ShapeDtypeStruct((N, 64), F32),
            jax.ShapeDtypeStruct((N, 128), F32),
            jax.ShapeDtypeStruct((8, 64), F32),
            jax.ShapeDtypeStruct((8, 32), F32),
            jax.ShapeDtypeStruct((8, 64), F32),
            jax.ShapeDtypeStruct((8, 64), F32),
        ],
    )(*args)


def _tc_decode(x_core, decW, decb, decg, decbe, qW, qb):
    def body(xc_r, decW_r, decb_r, decg_r, decbe_r, qW_r, qb_r, qs_r):
        h = jnp.maximum(_dot(xc_r[...], decW_r[...]) + decb_r[0:1, :], 0.0)
        nh = _ln(h, decg_r[0:1, :], decbe_r[0:1, :])
        qs_r[...] = _dot(nh, qW_r[...]) + qb_r[0:1, :]

    return pl.pallas_call(
        body,
        grid=(NBN,),
        in_specs=[
            pl.BlockSpec((BN, 64), lambda i: (i, 0)),
            _full((64, 32)), _full((8, 32)), _full((8, 32)), _full((8, 32)),
            _full((32, 2)), _full((8, 2)),
        ],
        out_specs=pl.BlockSpec((BN, 2), lambda i: (i, 0)),
        out_shape=jax.ShapeDtypeStruct((N, 2), F32),
    )(x_core, decW, decb, decg, decbe, qW, qb)


def kernel(x, edge_index, edge_attr, params):
    p = params
    src = edge_index[0].astype(jnp.int32)
    dst = edge_index[1].astype(jnp.int32)
    col3 = jnp.concatenate([dst, src]).reshape(NW, NOUT * NG, KIDX)

    def tile8(v):
        return jnp.tile(v.reshape(1, -1), (8, 1))

    W1 = p['em_W1']
    NW1 = p['nm_W1']
    GW = p['gm_W1']

    T0, NRc, vc, uenc, eb, nb = _tc_pre_node(
        x, p['ne_W'], tile8(p['ne_b']), tile8(p['ne_g']), tile8(p['ne_be']),
        W1[96:128], W1[192:224], NW1[0:32],
        tile8(p['ue_b']), tile8(p['ue_g']), tile8(p['ue_be']),
        tile8(p['em_b1']), W1[288:320], tile8(p['nm_b1']), NW1[160:192])
    EC = _tc_pre_edge(
        edge_attr, p['ee_W'], tile8(p['ee_b']), tile8(p['ee_g']),
        tile8(p['ee_be']), W1[0:32])

    zeros_init = jnp.zeros((NPAD, 64), F32)
    emb2 = tile8(p['em_b2'])
    emg = tile8(p['em_g'])
    embe = tile8(p['em_be'])
    nmb2 = tile8(p['nm_b2'])
    nmg = tile8(p['nm_g'])
    nmbe = tile8(p['nm_be'])

    e_core = None
    x_core = None
    u_core = None
    T = T0
    for r in range(4):
        G = _sc_gather(T, src, dst)
        e_core, esum = _tc_edge(
            e_core, EC, G, W1[32:96], p['em_W2'], eb, emb2, emg, embe,
            with_prev=(r > 0))
        agg2 = _sc_scatter(e_core.reshape(E2, 64), col3, zeros_init)
        x_core, T, vsum, u_core, eb, nb = _tc_node(
            x, NRc, x_core, agg2, NW1[32:96], NW1[96:160], p['nm_W2'], nb,
            nmb2, nmg, nmbe, T0, W1[128:192], W1[224:288],
            esum, vc, uenc, u_core, GW[0:32], GW[32:64], GW[64:128],
            GW[128:192], tile8(p['gm_b1']), p['gm_W2'], tile8(p['gm_b2']),
            tile8(p['gm_g']), tile8(p['gm_be']), tile8(p['em_b1']),
            W1[288:320], W1[320:352], tile8(p['nm_b1']), NW1[160:192],
            NW1[192:224], with_prev=(r > 0))

    qs = _tc_decode(x_core, p['dec_W'], tile8(p['dec_b']), tile8(p['dec_g']),
                    tile8(p['dec_be']), p['q_W'], tile8(p['q_b']))
    var_mask = x[:, 0] > x[:, 1]
    return qs, var_mask
